# baseline (speedup 1.0000x reference)
import functools

import jax
import jax.numpy as jnp
from jax import lax
from jax.experimental import pallas as pl
from jax.experimental.pallas import tpu as pltpu
from jax.experimental.pallas import tpu_sc as plsc

E = 8
K = 2
ALPHA = 1.702
LIMIT = 7.0

BM = 256
NB = 24
PCAP = NB * BM
SENT = 1 << 20

_INTERPRET = False


def _router_body(x_ref, rw_ref, ti_ref, tw_ref):
    x = x_ref[...]
    logits = jnp.dot(x, rw_ref[...], preferred_element_type=jnp.float32)
    t, e = logits.shape
    eio = jax.lax.broadcasted_iota(jnp.int32, (t, e), 1)
    m1 = jnp.max(logits, axis=1, keepdims=True)
    i1 = jnp.min(jnp.where(logits == m1, eio, e), axis=1, keepdims=True)
    masked = jnp.where(eio == i1, -jnp.inf, logits)
    m2 = jnp.max(masked, axis=1, keepdims=True)
    i2 = jnp.min(jnp.where(masked == m2, eio, e), axis=1, keepdims=True)
    w1 = 1.0 / (1.0 + jnp.exp(m2 - m1))
    ti_ref[...] = jnp.concatenate([i1, i2], axis=1)
    tw_ref[...] = jnp.concatenate([w1, 1.0 - w1], axis=1)


def _router(x, router_w):
    t = x.shape[0]
    return pl.pallas_call(
        _router_body,
        out_shape=(
            jax.ShapeDtypeStruct((t, K), jnp.int32),
            jax.ShapeDtypeStruct((t, K), jnp.float32),
        ),
        interpret=_INTERPRET,
    )(x, router_w)


def _rank_body(ti_ref, pos_ref, be_ref, ba_ref):
    t = ti_ref.shape[0]
    eio = jax.lax.broadcasted_iota(jnp.int32, (t, E), 1)
    oh0 = (ti_ref[:, 0:1] == eio).astype(jnp.float32)
    oh1 = (ti_ref[:, 1:2] == eio).astype(jnp.float32)
    ohsum = oh0 + oh1
    r0 = jax.lax.broadcasted_iota(jnp.int32, (t, t), 0)
    r1 = jax.lax.broadcasted_iota(jnp.int32, (t, t), 1)
    ltri = (r0 > r1).astype(jnp.bfloat16)
    cum = jnp.dot(ltri, ohsum.astype(jnp.bfloat16),
                  preferred_element_type=jnp.float32)
    rank0 = jnp.sum(cum * oh0, axis=1, keepdims=True)
    rank1 = jnp.sum((cum + oh0) * oh1, axis=1, keepdims=True)
    counts = jnp.sum(ohsum, axis=0, keepdims=True)
    padded = jnp.ceil(counts / BM) * BM
    u0 = jax.lax.broadcasted_iota(jnp.int32, (E, E), 0)
    u1 = jax.lax.broadcasted_iota(jnp.int32, (E, E), 1)
    utri = (u0 < u1).astype(jnp.float32)
    spad = jnp.dot(padded, utri, preferred_element_type=jnp.float32)
    pos0 = jnp.sum(spad * oh0, axis=1, keepdims=True) + rank0
    pos1 = jnp.sum(spad * oh1, axis=1, keepdims=True) + rank1
    pos_ref[...] = jnp.concatenate([pos0, pos1], axis=1).astype(jnp.int32)
    ends = spad + padded
    total = jnp.sum(padded)
    bs = (jax.lax.broadcasted_iota(jnp.int32, (8, 128), 1) * BM
          ).astype(jnp.float32)
    ex = jnp.zeros((8, 128), jnp.int32)
    for j in range(E):
        endj = jnp.sum(ends * (jax.lax.broadcasted_iota(
            jnp.int32, (1, E), 1) == j).astype(jnp.float32))
        ex = ex + (bs >= endj).astype(jnp.int32)
    be_ref[...] = jnp.minimum(ex, E - 1)
    ba_ref[...] = (bs < total).astype(jnp.int32)


def _rank(topk_i):
    t = topk_i.shape[0]
    return pl.pallas_call(
        _rank_body,
        out_shape=(
            jax.ShapeDtypeStruct((t, K), jnp.int32),
            jax.ShapeDtypeStruct((8, 128), jnp.int32),
            jax.ShapeDtypeStruct((8, 128), jnp.int32),
        ),
        interpret=_INTERPRET,
    )(topk_i)


def _scatter_sc_body(pos_hbm, wf_hbm, rows_out, ws_out,
                     pos_v, w_v, rows_v, ws_v):
    wid = lax.axis_index("c") * 16 + lax.axis_index("s")
    tk = pos_v.shape[0]
    cap = rows_v.shape[0]

    @pl.when(wid == 0)
    def _():
        pltpu.sync_copy(pos_hbm, pos_v)
        pltpu.sync_copy(wf_hbm, w_v)

        def prefill(i, _):
            i16 = pl.multiple_of(i * 16, 16)
            rows_v[pl.ds(i16, 16)] = jnp.full((16,), SENT, jnp.int32)
            ws_v[pl.ds(i16, 16)] = jnp.zeros((16,), jnp.float32)
            return 0
        lax.fori_loop(0, cap // 16, prefill, 0)

        def scan(i, _):
            i16 = pl.multiple_of(i * 16, 16)
            pv = pos_v[pl.ds(i16, 16)]
            wv = w_v[pl.ds(i16, 16)]
            halves = lax.iota(jnp.int32, 16) // K
            tvec = jnp.full((16,), i * (16 // K), jnp.int32) + halves
            plsc.store_scatter(rows_v, [pv], tvec)
            plsc.store_scatter(ws_v, [pv], wv)
            return 0
        lax.fori_loop(0, tk // 16, scan, 0)

        def copyout(ch, _):
            sl = pl.ds(pl.multiple_of(ch * BM, BM), BM)
            pltpu.sync_copy(rows_v.at[sl], rows_out.at[sl])
            pltpu.sync_copy(ws_v.at[sl], ws_out.at[sl])
            return 0
        lax.fori_loop(0, cap // BM, copyout, 0)


def _scatter_sc(pos_flat, wf_flat):
    tk = pos_flat.shape[0]
    fn = pl.kernel(
        _scatter_sc_body,
        compiler_params=pltpu.CompilerParams(needs_layout_passes=False),
        out_type=(
            jax.ShapeDtypeStruct((PCAP,), jnp.int32),
            jax.ShapeDtypeStruct((PCAP,), jnp.float32),
        ),
        mesh=plsc.VectorSubcoreMesh(core_axis_name="c", subcore_axis_name="s"),
        scratch_types=[
            pltpu.VMEM((tk,), jnp.int32),
            pltpu.VMEM((tk,), jnp.float32),
            pltpu.VMEM((PCAP,), jnp.int32),
            pltpu.VMEM((PCAP,), jnp.float32),
        ],
    )
    return fn(pos_flat, wf_flat)


_GROWS = 16
_GNBUF = 2


def _gather_sc_body(rows_hbm, x_hbm, xs_hbm, idxr_v, idxc0_v, idxc1_v,
                    buf0_v, buf1_v, gsem, wsem):
    wid = lax.axis_index("c") * 16 + lax.axis_index("s")
    t = x_hbm.shape[0]
    rows_per = xs_hbm.shape[0] // 32
    nch = rows_per // _GROWS
    bufs = (buf0_v, buf1_v)
    idxs = (idxc0_v, idxc1_v)

    def start_gather(ch):
        b = ch % _GNBUF
        off = pl.multiple_of(wid * rows_per + ch * _GROWS, 8)
        pltpu.sync_copy(rows_hbm.at[pl.ds(off, _GROWS)], idxr_v)
        idxs[b][...] = jnp.minimum(idxr_v[...], t - 1)
        return pltpu.async_copy(x_hbm.at[idxs[b]], bufs[b], gsem)

    def start_writeout(ch):
        b = ch % _GNBUF
        off = pl.multiple_of(wid * rows_per + ch * _GROWS, 8)
        return pltpu.async_copy(bufs[b], xs_hbm.at[pl.ds(off, _GROWS)], wsem)

    g = start_gather(0)
    wprev = None
    for ch in range(nch):
        g.wait()
        if wprev is not None:
            wprev.wait()
        w = start_writeout(ch)
        if ch + 1 < nch:
            g = start_gather(ch + 1)
        wprev = w
    wprev.wait()


def _gather_sc(row_ids, x):
    h = x.shape[1]
    fn = pl.kernel(
        _gather_sc_body,
        compiler_params=pltpu.CompilerParams(needs_layout_passes=False),
        out_type=jax.ShapeDtypeStruct((PCAP, h), jnp.float32),
        mesh=plsc.VectorSubcoreMesh(core_axis_name="c", subcore_axis_name="s"),
        scratch_types=[
            pltpu.VMEM((_GROWS,), jnp.int32),
            pltpu.VMEM((_GROWS,), jnp.int32),
            pltpu.VMEM((_GROWS,), jnp.int32),
            pltpu.VMEM((_GROWS, h), jnp.float32),
            pltpu.VMEM((_GROWS, h), jnp.float32),
            pltpu.SemaphoreType.DMA,
            pltpu.SemaphoreType.DMA,
        ],
    )
    return fn(row_ids, x)


def _dispatch(topk_i, topk_w, x):
    t = x.shape[0]
    ef = topk_i.reshape(-1)
    wf = topk_w.reshape(-1)
    tk = ef.shape[0]
    counts = jnp.bincount(ef, length=E)
    padded = ((counts + BM - 1) // BM) * BM
    s_pad = jnp.concatenate([jnp.zeros((1,), jnp.int32),
                             jnp.cumsum(padded)[:-1].astype(jnp.int32)])
    s_cmp = jnp.concatenate([jnp.zeros((1,), jnp.int32),
                             jnp.cumsum(counts)[:-1].astype(jnp.int32)])
    order = jnp.argsort(ef, stable=True)
    es = ef[order]
    rank = jnp.arange(tk, dtype=jnp.int32) - s_cmp[es]
    pos = s_pad[es] + rank
    row_ids = jnp.full((PCAP,), SENT, jnp.int32).at[pos].set(
        (order // K).astype(jnp.int32))
    w_sorted = jnp.zeros((PCAP,), jnp.float32).at[pos].set(wf[order])
    ends = (s_pad + padded.astype(jnp.int32))
    bstart = jnp.arange(32, dtype=jnp.int32) * BM
    block_expert = jnp.minimum(
        jnp.sum(bstart[:, None] >= ends[None, :], axis=1), E - 1
    ).astype(jnp.int32)
    total = jnp.sum(padded).astype(jnp.int32)
    block_active = (bstart < total).astype(jnp.int32)
    x_sorted = x[jnp.minimum(row_ids, t - 1)]
    return x_sorted, row_ids, w_sorted, block_expert, block_active


def _mlp1_body(be_ref, ba_ref, x_ref, w_ref, act_ref):
    m = pl.program_id(0)

    @pl.when(ba_ref[m] == 1)
    def _():
        x = x_ref[...].astype(jnp.bfloat16)
        w = w_ref[0].astype(jnp.bfloat16)
        gu = jnp.dot(x, w, preferred_element_type=jnp.float32)
        gate = jnp.minimum(gu, LIMIT)
        up = jnp.clip(jnp.roll(gu, -1, axis=1), -LIMIT, LIMIT)
        glu = gate / (1.0 + jnp.exp(-ALPHA * gate))
        act_i = (up + 1.0) * glu
        n2 = gu.shape[1]
        sel = (jax.lax.broadcasted_iota(jnp.int32, (n2, n2 // 2), 0)
               == 2 * jax.lax.broadcasted_iota(jnp.int32, (n2, n2 // 2), 1)
               ).astype(jnp.bfloat16)
        act_ref[...] = jnp.dot(act_i.astype(jnp.bfloat16), sel,
                               preferred_element_type=jnp.float32
                               ).astype(jnp.bfloat16)

    @pl.when(ba_ref[m] == 0)
    def _():
        act_ref[...] = jnp.zeros_like(act_ref)


def _mlp1(x_sorted, gate_up, block_expert, block_active):
    h = x_sorted.shape[1]
    f2 = gate_up.shape[2]
    f = f2 // 2
    fb = 512
    nf = f // fb
    grid = (NB, nf)
    return pl.pallas_call(
        _mlp1_body,
        grid_spec=pltpu.PrefetchScalarGridSpec(
            num_scalar_prefetch=2,
            grid=grid,
            in_specs=[
                pl.BlockSpec((BM, h), lambda m, fi, be, ba: (m, 0)),
                pl.BlockSpec((1, h, 2 * fb),
                             lambda m, fi, be, ba: (be[m], 0, fi)),
            ],
            out_specs=pl.BlockSpec((BM, fb), lambda m, fi, be, ba: (m, fi)),
        ),
        out_shape=jax.ShapeDtypeStruct((PCAP, f), jnp.bfloat16),
        interpret=_INTERPRET,
    )(block_expert, block_active, x_sorted, gate_up)


def _mlp2_body(be_ref, ba_ref, a_ref, w_ref, y_ref):
    m = pl.program_id(0)

    @pl.when(ba_ref[m] == 1)
    def _():
        y_ref[...] = jnp.dot(a_ref[...], w_ref[0].astype(jnp.bfloat16),
                             preferred_element_type=jnp.float32)

    @pl.when(ba_ref[m] == 0)
    def _():
        y_ref[...] = jnp.zeros_like(y_ref)


def _mlp2(act, down, block_expert, block_active):
    f = act.shape[1]
    h = down.shape[2]
    hb = 1024
    nh = h // hb
    grid = (NB, nh)
    return pl.pallas_call(
        _mlp2_body,
        grid_spec=pltpu.PrefetchScalarGridSpec(
            num_scalar_prefetch=2,
            grid=grid,
            in_specs=[
                pl.BlockSpec((BM, f), lambda m, hi, be, ba: (m, 0)),
                pl.BlockSpec((1, f, hb),
                             lambda m, hi, be, ba: (be[m], 0, hi)),
            ],
            out_specs=pl.BlockSpec((BM, hb), lambda m, hi, be, ba: (m, hi)),
        ),
        out_shape=jax.ShapeDtypeStruct((PCAP, h), jnp.float32),
        interpret=_INTERPRET,
    )(block_expert, block_active, act, down)


def _combine_body(ids_ref, w_ref, y_ref, out_ref):
    tb = pl.program_id(0)
    rb = pl.program_id(1)

    @pl.when(rb == 0)
    def _():
        out_ref[...] = jnp.zeros_like(out_ref)

    ids = ids_ref[0, 0, :]
    w = w_ref[0, 0, :]
    tio = jax.lax.broadcasted_iota(jnp.int32, (BM, BM), 0) + tb * BM
    q = jnp.where(ids[None, :] == tio, w[None, :], 0.0).astype(jnp.bfloat16)
    out_ref[...] += jnp.dot(q, y_ref[...].astype(jnp.bfloat16),
                            preferred_element_type=jnp.float32)


def _combine(row_ids, w_sorted, y, t):
    h = y.shape[1]
    nt = t // BM
    grid = (nt, NB)
    return pl.pallas_call(
        _combine_body,
        grid=grid,
        in_specs=[
            pl.BlockSpec((1, 1, BM), lambda ti, ri: (ri, 0, 0)),
            pl.BlockSpec((1, 1, BM), lambda ti, ri: (ri, 0, 0)),
            pl.BlockSpec((BM, h), lambda ti, ri: (ri, 0)),
        ],
        out_specs=pl.BlockSpec((BM, h), lambda ti, ri: (ti, 0)),
        out_shape=jax.ShapeDtypeStruct((t, h), jnp.float32),
        interpret=_INTERPRET,
    )(row_ids.reshape(NB, 1, BM), w_sorted.reshape(NB, 1, BM), y)


def kernel(hidden_states, router_w, gate_up_proj, down_proj):
    b, s, h = hidden_states.shape
    t = b * s
    x = hidden_states.reshape(t, h)
    topk_i, topk_w = _router(x, router_w)
    pos, be_tbl, ba_tbl = _rank(topk_i)
    row_ids, w_sorted = _scatter_sc(pos.reshape(-1), topk_w.reshape(-1))
    be = be_tbl.reshape(-1)[:32]
    ba = ba_tbl.reshape(-1)[:32]
    x_sorted = _gather_sc(row_ids, x)
    act = _mlp1(x_sorted, gate_up_proj, be, ba)
    y = _mlp2(act, down_proj, be, ba)
    out = _combine(row_ids, w_sorted, y, t)
    return out.reshape(b, s, h)


# speedup vs baseline: 1.2169x; 1.2169x over previous
import functools

import jax
import jax.numpy as jnp
from jax import lax
from jax.experimental import pallas as pl
from jax.experimental.pallas import tpu as pltpu
from jax.experimental.pallas import tpu_sc as plsc

E = 8
K = 2
ALPHA = 1.702
LIMIT = 7.0

BM = 256
NB = 24
PCAP = NB * BM
SENT = 1 << 20

_INTERPRET = False


def _router_body(x_ref, rw_ref, ti_ref, tw_ref):
    x = x_ref[...]
    logits = jnp.dot(x, rw_ref[...], preferred_element_type=jnp.float32)
    t, e = logits.shape
    eio = jax.lax.broadcasted_iota(jnp.int32, (t, e), 1)
    m1 = jnp.max(logits, axis=1, keepdims=True)
    i1 = jnp.min(jnp.where(logits == m1, eio, e), axis=1, keepdims=True)
    masked = jnp.where(eio == i1, -jnp.inf, logits)
    m2 = jnp.max(masked, axis=1, keepdims=True)
    i2 = jnp.min(jnp.where(masked == m2, eio, e), axis=1, keepdims=True)
    w1 = 1.0 / (1.0 + jnp.exp(m2 - m1))
    ti_ref[...] = jnp.concatenate([i1, i2], axis=1)
    tw_ref[...] = jnp.concatenate([w1, 1.0 - w1], axis=1)


def _router(x, router_w):
    t = x.shape[0]
    return pl.pallas_call(
        _router_body,
        out_shape=(
            jax.ShapeDtypeStruct((t, K), jnp.int32),
            jax.ShapeDtypeStruct((t, K), jnp.float32),
        ),
        interpret=_INTERPRET,
    )(x, router_w)


def _rank_body(ti_ref, pos_ref, be_ref, ba_ref):
    t = ti_ref.shape[0]
    eio = jax.lax.broadcasted_iota(jnp.int32, (t, E), 1)
    oh0 = (ti_ref[:, 0:1] == eio).astype(jnp.float32)
    oh1 = (ti_ref[:, 1:2] == eio).astype(jnp.float32)
    ohsum = oh0 + oh1
    r0 = jax.lax.broadcasted_iota(jnp.int32, (t, t), 0)
    r1 = jax.lax.broadcasted_iota(jnp.int32, (t, t), 1)
    ltri = (r0 > r1).astype(jnp.bfloat16)
    cum = jnp.dot(ltri, ohsum.astype(jnp.bfloat16),
                  preferred_element_type=jnp.float32)
    rank0 = jnp.sum(cum * oh0, axis=1, keepdims=True)
    rank1 = jnp.sum((cum + oh0) * oh1, axis=1, keepdims=True)
    counts = jnp.sum(ohsum, axis=0, keepdims=True)
    padded = jnp.ceil(counts / BM) * BM
    u0 = jax.lax.broadcasted_iota(jnp.int32, (E, E), 0)
    u1 = jax.lax.broadcasted_iota(jnp.int32, (E, E), 1)
    utri = (u0 < u1).astype(jnp.float32)
    spad = jnp.dot(padded, utri, preferred_element_type=jnp.float32)
    pos0 = jnp.sum(spad * oh0, axis=1, keepdims=True) + rank0
    pos1 = jnp.sum(spad * oh1, axis=1, keepdims=True) + rank1
    pos_ref[...] = jnp.concatenate([pos0, pos1], axis=1).astype(jnp.int32)
    ends = spad + padded
    total = jnp.sum(padded)
    bs = (jax.lax.broadcasted_iota(jnp.int32, (8, 128), 1) * BM
          ).astype(jnp.float32)
    ex = jnp.zeros((8, 128), jnp.int32)
    for j in range(E):
        endj = jnp.sum(ends * (jax.lax.broadcasted_iota(
            jnp.int32, (1, E), 1) == j).astype(jnp.float32))
        ex = ex + (bs >= endj).astype(jnp.int32)
    be_ref[...] = jnp.minimum(ex, E - 1)
    ba_ref[...] = (bs < total).astype(jnp.int32)


def _rank(topk_i):
    t = topk_i.shape[0]
    return pl.pallas_call(
        _rank_body,
        out_shape=(
            jax.ShapeDtypeStruct((t, K), jnp.int32),
            jax.ShapeDtypeStruct((8, 128), jnp.int32),
            jax.ShapeDtypeStruct((8, 128), jnp.int32),
        ),
        interpret=_INTERPRET,
    )(topk_i)


def _scatter_sc_body(pos_hbm, wf_hbm, rows_out, ws_out,
                     pos_v, w_v, rows_v, ws_v):
    wid = lax.axis_index("c") * 16 + lax.axis_index("s")
    tk = pos_v.shape[0]
    cap = rows_v.shape[0]

    @pl.when(wid == 0)
    def _():
        pltpu.sync_copy(pos_hbm, pos_v)
        pltpu.sync_copy(wf_hbm, w_v)

        def prefill(i, _):
            i16 = pl.multiple_of(i * 16, 16)
            rows_v[pl.ds(i16, 16)] = jnp.full((16,), SENT, jnp.int32)
            ws_v[pl.ds(i16, 16)] = jnp.zeros((16,), jnp.float32)
            return 0
        lax.fori_loop(0, cap // 16, prefill, 0)

        def scan(i, _):
            i16 = pl.multiple_of(i * 16, 16)
            pv = pos_v[pl.ds(i16, 16)]
            wv = w_v[pl.ds(i16, 16)]
            halves = lax.iota(jnp.int32, 16) // K
            tvec = jnp.full((16,), i * (16 // K), jnp.int32) + halves
            plsc.store_scatter(rows_v, [pv], tvec)
            plsc.store_scatter(ws_v, [pv], wv)
            return 0
        lax.fori_loop(0, tk // 16, scan, 0)

        def copyout(ch, _):
            sl = pl.ds(pl.multiple_of(ch * BM, BM), BM)
            pltpu.sync_copy(rows_v.at[sl], rows_out.at[sl])
            pltpu.sync_copy(ws_v.at[sl], ws_out.at[sl])
            return 0
        lax.fori_loop(0, cap // BM, copyout, 0)


def _scatter_sc(pos_flat, wf_flat):
    tk = pos_flat.shape[0]
    fn = pl.kernel(
        _scatter_sc_body,
        compiler_params=pltpu.CompilerParams(needs_layout_passes=False),
        out_type=(
            jax.ShapeDtypeStruct((PCAP,), jnp.int32),
            jax.ShapeDtypeStruct((PCAP,), jnp.float32),
        ),
        mesh=plsc.VectorSubcoreMesh(core_axis_name="c", subcore_axis_name="s"),
        scratch_types=[
            pltpu.VMEM((tk,), jnp.int32),
            pltpu.VMEM((tk,), jnp.float32),
            pltpu.VMEM((PCAP,), jnp.int32),
            pltpu.VMEM((PCAP,), jnp.float32),
        ],
    )
    return fn(pos_flat, wf_flat)


def _dispatch(topk_i, topk_w, x):
    t = x.shape[0]
    ef = topk_i.reshape(-1)
    wf = topk_w.reshape(-1)
    tk = ef.shape[0]
    counts = jnp.bincount(ef, length=E)
    padded = ((counts + BM - 1) // BM) * BM
    s_pad = jnp.concatenate([jnp.zeros((1,), jnp.int32),
                             jnp.cumsum(padded)[:-1].astype(jnp.int32)])
    s_cmp = jnp.concatenate([jnp.zeros((1,), jnp.int32),
                             jnp.cumsum(counts)[:-1].astype(jnp.int32)])
    order = jnp.argsort(ef, stable=True)
    es = ef[order]
    rank = jnp.arange(tk, dtype=jnp.int32) - s_cmp[es]
    pos = s_pad[es] + rank
    row_ids = jnp.full((PCAP,), SENT, jnp.int32).at[pos].set(
        (order // K).astype(jnp.int32))
    w_sorted = jnp.zeros((PCAP,), jnp.float32).at[pos].set(wf[order])
    ends = (s_pad + padded.astype(jnp.int32))
    bstart = jnp.arange(32, dtype=jnp.int32) * BM
    block_expert = jnp.minimum(
        jnp.sum(bstart[:, None] >= ends[None, :], axis=1), E - 1
    ).astype(jnp.int32)
    total = jnp.sum(padded).astype(jnp.int32)
    block_active = (bstart < total).astype(jnp.int32)
    x_sorted = x[jnp.minimum(row_ids, t - 1)]
    return x_sorted, row_ids, w_sorted, block_expert, block_active


def _mlp1_body(be_ref, ba_ref, x_ref, ids_ref, w_ref, act_ref, xs_s):
    m = pl.program_id(0)
    f = pl.program_id(1)

    @pl.when((ba_ref[m] == 1) & (f == 0))
    def _():
        t = x_ref.shape[0]
        ids = ids_ref[0]
        tio = jax.lax.broadcasted_iota(jnp.int32, (ids.shape[0], t), 1)
        p = (ids == tio).astype(jnp.bfloat16)
        xs_s[...] = jnp.dot(p, x_ref[...],
                            preferred_element_type=jnp.float32
                            ).astype(jnp.bfloat16)

    @pl.when(ba_ref[m] == 1)
    def _():
        w = w_ref[0].astype(jnp.bfloat16)
        gu = jnp.dot(xs_s[...], w, preferred_element_type=jnp.float32)
        gate = jnp.minimum(gu, LIMIT)
        up = jnp.clip(jnp.roll(gu, -1, axis=1), -LIMIT, LIMIT)
        glu = gate / (1.0 + jnp.exp(-ALPHA * gate))
        act_i = (up + 1.0) * glu
        n2 = gu.shape[1]
        sel = (jax.lax.broadcasted_iota(jnp.int32, (n2, n2 // 2), 0)
               == 2 * jax.lax.broadcasted_iota(jnp.int32, (n2, n2 // 2), 1)
               ).astype(jnp.bfloat16)
        act_ref[...] = jnp.dot(act_i.astype(jnp.bfloat16), sel,
                               preferred_element_type=jnp.float32
                               ).astype(jnp.bfloat16)

    @pl.when(ba_ref[m] == 0)
    def _():
        act_ref[...] = jnp.zeros_like(act_ref)


def _mlp1(x_bf, row_ids_2d, gate_up, block_expert, block_active):
    t, h = x_bf.shape
    f2 = gate_up.shape[2]
    f = f2 // 2
    fb = 512
    nf = f // fb
    grid = (NB, nf)
    return pl.pallas_call(
        _mlp1_body,
        grid_spec=pltpu.PrefetchScalarGridSpec(
            num_scalar_prefetch=2,
            grid=grid,
            in_specs=[
                pl.BlockSpec((t, h), lambda m, fi, be, ba: (0, 0)),
                pl.BlockSpec((1, BM, 1), lambda m, fi, be, ba: (m, 0, 0)),
                pl.BlockSpec((1, h, 2 * fb),
                             lambda m, fi, be, ba: (be[m], 0, fi)),
            ],
            out_specs=pl.BlockSpec((BM, fb), lambda m, fi, be, ba: (m, fi)),
            scratch_shapes=[pltpu.VMEM((BM, h), jnp.bfloat16)],
        ),
        out_shape=jax.ShapeDtypeStruct((PCAP, f), jnp.bfloat16),
        interpret=_INTERPRET,
    )(block_expert, block_active, x_bf, row_ids_2d, gate_up)


def _mlp2_body(be_ref, ba_ref, a_ref, w_ref, y_ref):
    m = pl.program_id(0)

    @pl.when(ba_ref[m] == 1)
    def _():
        y_ref[...] = jnp.dot(a_ref[...], w_ref[0].astype(jnp.bfloat16),
                             preferred_element_type=jnp.float32)

    @pl.when(ba_ref[m] == 0)
    def _():
        y_ref[...] = jnp.zeros_like(y_ref)


def _mlp2(act, down, block_expert, block_active):
    f = act.shape[1]
    h = down.shape[2]
    hb = 1024
    nh = h // hb
    grid = (NB, nh)
    return pl.pallas_call(
        _mlp2_body,
        grid_spec=pltpu.PrefetchScalarGridSpec(
            num_scalar_prefetch=2,
            grid=grid,
            in_specs=[
                pl.BlockSpec((BM, f), lambda m, hi, be, ba: (m, 0)),
                pl.BlockSpec((1, f, hb),
                             lambda m, hi, be, ba: (be[m], 0, hi)),
            ],
            out_specs=pl.BlockSpec((BM, hb), lambda m, hi, be, ba: (m, hi)),
        ),
        out_shape=jax.ShapeDtypeStruct((PCAP, h), jnp.float32),
        interpret=_INTERPRET,
    )(block_expert, block_active, act, down)


def _combine_body(ids_ref, w_ref, y_ref, out_ref):
    tb = pl.program_id(0)
    rb = pl.program_id(1)

    @pl.when(rb == 0)
    def _():
        out_ref[...] = jnp.zeros_like(out_ref)

    ids = ids_ref[0, 0, :]
    w = w_ref[0, 0, :]
    tio = jax.lax.broadcasted_iota(jnp.int32, (BM, BM), 0) + tb * BM
    q = jnp.where(ids[None, :] == tio, w[None, :], 0.0).astype(jnp.bfloat16)
    out_ref[...] += jnp.dot(q, y_ref[...].astype(jnp.bfloat16),
                            preferred_element_type=jnp.float32)


def _combine(row_ids, w_sorted, y, t):
    h = y.shape[1]
    nt = t // BM
    grid = (nt, NB)
    return pl.pallas_call(
        _combine_body,
        grid=grid,
        in_specs=[
            pl.BlockSpec((1, 1, BM), lambda ti, ri: (ri, 0, 0)),
            pl.BlockSpec((1, 1, BM), lambda ti, ri: (ri, 0, 0)),
            pl.BlockSpec((BM, h), lambda ti, ri: (ri, 0)),
        ],
        out_specs=pl.BlockSpec((BM, h), lambda ti, ri: (ti, 0)),
        out_shape=jax.ShapeDtypeStruct((t, h), jnp.float32),
        interpret=_INTERPRET,
    )(row_ids.reshape(NB, 1, BM), w_sorted.reshape(NB, 1, BM), y)


def kernel(hidden_states, router_w, gate_up_proj, down_proj):
    b, s, h = hidden_states.shape
    t = b * s
    x = hidden_states.reshape(t, h)
    topk_i, topk_w = _router(x, router_w)
    pos, be_tbl, ba_tbl = _rank(topk_i)
    row_ids, w_sorted = _scatter_sc(pos.reshape(-1), topk_w.reshape(-1))
    be = be_tbl.reshape(-1)[:32]
    ba = ba_tbl.reshape(-1)[:32]
    act = _mlp1(x.astype(jnp.bfloat16), row_ids.reshape(NB, BM, 1),
                gate_up_proj, be, ba)
    y = _mlp2(act, down_proj, be, ba)
    out = _combine(row_ids, w_sorted, y, t)
    return out.reshape(b, s, h)


# speedup vs baseline: 1.2171x; 1.0002x over previous
import jax
import jax.numpy as jnp
from jax import lax
from jax.experimental import pallas as pl
from jax.experimental.pallas import tpu as pltpu
from jax.experimental.pallas import tpu_sc as plsc

E = 8
K = 2
ALPHA = 1.702
LIMIT = 7.0

BM = 256
NB = 24
PCAP = NB * BM
SENT = 1 << 20

_INTERPRET = False


def _router_body(x_ref, rw_ref, ti_ref, tw_ref):
    x = x_ref[...]
    logits = jnp.dot(x, rw_ref[...], preferred_element_type=jnp.float32)
    t, e = logits.shape
    eio = jax.lax.broadcasted_iota(jnp.int32, (t, e), 1)
    m1 = jnp.max(logits, axis=1, keepdims=True)
    i1 = jnp.min(jnp.where(logits == m1, eio, e), axis=1, keepdims=True)
    masked = jnp.where(eio == i1, -jnp.inf, logits)
    m2 = jnp.max(masked, axis=1, keepdims=True)
    i2 = jnp.min(jnp.where(masked == m2, eio, e), axis=1, keepdims=True)
    w1 = 1.0 / (1.0 + jnp.exp(m2 - m1))
    ti_ref[...] = jnp.concatenate([i1, i2], axis=1)
    tw_ref[...] = jnp.concatenate([w1, 1.0 - w1], axis=1)


def _router(x, router_w):
    t = x.shape[0]
    return pl.pallas_call(
        _router_body,
        out_shape=(
            jax.ShapeDtypeStruct((t, K), jnp.int32),
            jax.ShapeDtypeStruct((t, K), jnp.float32),
        ),
        interpret=_INTERPRET,
    )(x, router_w)


def _rank_body(ti_ref, pos_ref, be_ref, ba_ref):
    t = ti_ref.shape[0]
    eio = jax.lax.broadcasted_iota(jnp.int32, (t, E), 1)
    oh0 = (ti_ref[:, 0:1] == eio).astype(jnp.float32)
    oh1 = (ti_ref[:, 1:2] == eio).astype(jnp.float32)
    ohsum = oh0 + oh1
    r0 = jax.lax.broadcasted_iota(jnp.int32, (t, t), 0)
    r1 = jax.lax.broadcasted_iota(jnp.int32, (t, t), 1)
    ltri = (r0 > r1).astype(jnp.bfloat16)
    cum = jnp.dot(ltri, ohsum.astype(jnp.bfloat16),
                  preferred_element_type=jnp.float32)
    rank0 = jnp.sum(cum * oh0, axis=1, keepdims=True)
    rank1 = jnp.sum((cum + oh0) * oh1, axis=1, keepdims=True)
    counts = jnp.sum(ohsum, axis=0, keepdims=True)
    padded = jnp.ceil(counts / BM) * BM
    u0 = jax.lax.broadcasted_iota(jnp.int32, (E, E), 0)
    u1 = jax.lax.broadcasted_iota(jnp.int32, (E, E), 1)
    utri = (u0 < u1).astype(jnp.float32)
    spad = jnp.dot(padded, utri, preferred_element_type=jnp.float32)
    pos0 = jnp.sum(spad * oh0, axis=1, keepdims=True) + rank0
    pos1 = jnp.sum(spad * oh1, axis=1, keepdims=True) + rank1
    pos_ref[...] = jnp.concatenate([pos0, pos1], axis=1).astype(jnp.int32)
    ends = spad + padded
    total = jnp.sum(padded)
    bs = (jax.lax.broadcasted_iota(jnp.int32, (8, 128), 1) * BM
          ).astype(jnp.float32)
    ex = jnp.zeros((8, 128), jnp.int32)
    for j in range(E):
        endj = jnp.sum(ends * (jax.lax.broadcasted_iota(
            jnp.int32, (1, E), 1) == j).astype(jnp.float32))
        ex = ex + (bs >= endj).astype(jnp.int32)
    be_ref[...] = jnp.minimum(ex, E - 1)
    ba_ref[...] = (bs < total).astype(jnp.int32)


def _rank(topk_i):
    t = topk_i.shape[0]
    return pl.pallas_call(
        _rank_body,
        out_shape=(
            jax.ShapeDtypeStruct((t, K), jnp.int32),
            jax.ShapeDtypeStruct((8, 128), jnp.int32),
            jax.ShapeDtypeStruct((8, 128), jnp.int32),
        ),
        interpret=_INTERPRET,
    )(topk_i)


def _scatter_sc_body(pos_hbm, wf_hbm, rows_out, ws_out,
                     pos_v, w_v, rows_v, ws_v):
    wid = lax.axis_index("c") * 16 + lax.axis_index("s")
    tk = pos_v.shape[0]
    cap = rows_v.shape[0]

    @pl.when(wid == 0)
    def _():
        pltpu.sync_copy(pos_hbm, pos_v)
        pltpu.sync_copy(wf_hbm, w_v)

        def prefill(i, _):
            i16 = pl.multiple_of(i * 16, 16)
            rows_v[pl.ds(i16, 16)] = jnp.full((16,), SENT, jnp.int32)
            ws_v[pl.ds(i16, 16)] = jnp.zeros((16,), jnp.float32)
            return 0
        lax.fori_loop(0, cap // 16, prefill, 0)

        def scan(i, _):
            i16 = pl.multiple_of(i * 16, 16)
            pv = pos_v[pl.ds(i16, 16)]
            wv = w_v[pl.ds(i16, 16)]
            halves = lax.iota(jnp.int32, 16) // K
            tvec = jnp.full((16,), i * (16 // K), jnp.int32) + halves
            plsc.store_scatter(rows_v, [pv], tvec)
            plsc.store_scatter(ws_v, [pv], wv)
            return 0
        lax.fori_loop(0, tk // 16, scan, 0)

        def copyout(ch, _):
            sl = pl.ds(pl.multiple_of(ch * BM, BM), BM)
            pltpu.sync_copy(rows_v.at[sl], rows_out.at[sl])
            pltpu.sync_copy(ws_v.at[sl], ws_out.at[sl])
            return 0
        lax.fori_loop(0, cap // BM, copyout, 0)


def _scatter_sc(pos_flat, wf_flat):
    tk = pos_flat.shape[0]
    fn = pl.kernel(
        _scatter_sc_body,
        compiler_params=pltpu.CompilerParams(needs_layout_passes=False),
        out_type=(
            jax.ShapeDtypeStruct((PCAP,), jnp.int32),
            jax.ShapeDtypeStruct((PCAP,), jnp.float32),
        ),
        mesh=plsc.VectorSubcoreMesh(core_axis_name="c", subcore_axis_name="s"),
        scratch_types=[
            pltpu.VMEM((tk,), jnp.int32),
            pltpu.VMEM((tk,), jnp.float32),
            pltpu.VMEM((PCAP,), jnp.int32),
            pltpu.VMEM((PCAP,), jnp.float32),
        ],
    )
    return fn(pos_flat, wf_flat)


def _mlp1_body(be_ref, ba_ref, x_ref, ids_ref, w_ref, act_ref, xs_s):
    m = pl.program_id(0)
    f = pl.program_id(1)

    @pl.when((ba_ref[m] == 1) & (f == 0))
    def _():
        t = x_ref.shape[0]
        ids = ids_ref[0]
        tio = jax.lax.broadcasted_iota(jnp.int32, (ids.shape[0], t), 1)
        p = (ids == tio).astype(jnp.bfloat16)
        xs_s[...] = jnp.dot(p, x_ref[...],
                            preferred_element_type=jnp.float32
                            ).astype(jnp.bfloat16)

    @pl.when(ba_ref[m] == 1)
    def _():
        w = w_ref[0].astype(jnp.bfloat16)
        gu = jnp.dot(xs_s[...], w, preferred_element_type=jnp.float32)
        gate = jnp.minimum(gu, LIMIT)
        up = jnp.clip(jnp.roll(gu, -1, axis=1), -LIMIT, LIMIT)
        glu = gate / (1.0 + jnp.exp(-ALPHA * gate))
        act_i = (up + 1.0) * glu
        n2 = gu.shape[1]
        sel = (jax.lax.broadcasted_iota(jnp.int32, (n2, n2 // 2), 0)
               == 2 * jax.lax.broadcasted_iota(jnp.int32, (n2, n2 // 2), 1)
               ).astype(jnp.bfloat16)
        act_ref[...] = jnp.dot(act_i.astype(jnp.bfloat16), sel,
                               preferred_element_type=jnp.float32
                               ).astype(jnp.bfloat16)

    @pl.when(ba_ref[m] == 0)
    def _():
        act_ref[...] = jnp.zeros_like(act_ref)


def _mlp1(x_bf, row_ids_2d, gate_up, block_expert, block_active):
    t, h = x_bf.shape
    f2 = gate_up.shape[2]
    f = f2 // 2
    fb = 512
    nf = f // fb
    grid = (NB, nf)
    return pl.pallas_call(
        _mlp1_body,
        grid_spec=pltpu.PrefetchScalarGridSpec(
            num_scalar_prefetch=2,
            grid=grid,
            in_specs=[
                pl.BlockSpec((t, h), lambda m, fi, be, ba: (0, 0)),
                pl.BlockSpec((1, BM, 1), lambda m, fi, be, ba: (m, 0, 0)),
                pl.BlockSpec((1, h, 2 * fb),
                             lambda m, fi, be, ba: (be[m], 0, fi)),
            ],
            out_specs=pl.BlockSpec((BM, fb), lambda m, fi, be, ba: (m, fi)),
            scratch_shapes=[pltpu.VMEM((BM, h), jnp.bfloat16)],
        ),
        out_shape=jax.ShapeDtypeStruct((PCAP, f), jnp.bfloat16),
        interpret=_INTERPRET,
    )(block_expert, block_active, x_bf, row_ids_2d, gate_up)


def _mlp2_body(be_ref, ba_ref, a_ref, w_ref, y_ref):
    m = pl.program_id(0)

    @pl.when(ba_ref[m] == 1)
    def _():
        y_ref[...] = jnp.dot(a_ref[...], w_ref[0].astype(jnp.bfloat16),
                             preferred_element_type=jnp.float32)

    @pl.when(ba_ref[m] == 0)
    def _():
        y_ref[...] = jnp.zeros_like(y_ref)


def _mlp2(act, down, block_expert, block_active):
    f = act.shape[1]
    h = down.shape[2]
    hb = 1024
    nh = h // hb
    grid = (NB, nh)
    return pl.pallas_call(
        _mlp2_body,
        grid_spec=pltpu.PrefetchScalarGridSpec(
            num_scalar_prefetch=2,
            grid=grid,
            in_specs=[
                pl.BlockSpec((BM, f), lambda m, hi, be, ba: (m, 0)),
                pl.BlockSpec((1, f, hb),
                             lambda m, hi, be, ba: (be[m], 0, hi)),
            ],
            out_specs=pl.BlockSpec((BM, hb), lambda m, hi, be, ba: (m, hi)),
        ),
        out_shape=jax.ShapeDtypeStruct((PCAP, h), jnp.float32),
        interpret=_INTERPRET,
    )(block_expert, block_active, act, down)


def _combine_body(ids_ref, w_ref, y_ref, out_ref):
    tb = pl.program_id(0)
    rb = pl.program_id(1)

    @pl.when(rb == 0)
    def _():
        out_ref[...] = jnp.zeros_like(out_ref)

    ids = ids_ref[0, 0, :]
    w = w_ref[0, 0, :]
    tio = jax.lax.broadcasted_iota(jnp.int32, (BM, BM), 0) + tb * BM
    q = jnp.where(ids[None, :] == tio, w[None, :], 0.0).astype(jnp.bfloat16)
    out_ref[...] += jnp.dot(q, y_ref[...].astype(jnp.bfloat16),
                            preferred_element_type=jnp.float32)


def _combine(row_ids, w_sorted, y, t):
    h = y.shape[1]
    nt = t // BM
    grid = (nt, NB)
    return pl.pallas_call(
        _combine_body,
        grid=grid,
        in_specs=[
            pl.BlockSpec((1, 1, BM), lambda ti, ri: (ri, 0, 0)),
            pl.BlockSpec((1, 1, BM), lambda ti, ri: (ri, 0, 0)),
            pl.BlockSpec((BM, h), lambda ti, ri: (ri, 0)),
        ],
        out_specs=pl.BlockSpec((BM, h), lambda ti, ri: (ti, 0)),
        out_shape=jax.ShapeDtypeStruct((t, h), jnp.float32),
        interpret=_INTERPRET,
    )(row_ids.reshape(NB, 1, BM), w_sorted.reshape(NB, 1, BM), y)


def kernel(hidden_states, router_w, gate_up_proj, down_proj):
    b, s, h = hidden_states.shape
    t = b * s
    x = hidden_states.reshape(t, h)
    topk_i, topk_w = _router(x, router_w)
    pos, be_tbl, ba_tbl = _rank(topk_i)
    row_ids, w_sorted = _scatter_sc(pos.reshape(-1), topk_w.reshape(-1))
    be = be_tbl.reshape(-1)[:32]
    ba = ba_tbl.reshape(-1)[:32]
    act = _mlp1(x.astype(jnp.bfloat16), row_ids.reshape(NB, BM, 1),
                gate_up_proj, be, ba)
    y = _mlp2(act, down_proj, be, ba)
    out = _combine(row_ids, w_sorted, y, t)
    return out.reshape(b, s, h)
